# baseline (speedup 1.0000x reference)
import jax
import jax.numpy as jnp
from jax.experimental import pallas as pl

N = 50000
E = 800000
G = 256
K = 4
H = 32


def _head_body(gfeat_ref, wl_ref, bl_ref, out_ref):
    out = jnp.dot(gfeat_ref[...], wl_ref[...],
                  preferred_element_type=jnp.float32) + bl_ref[...]
    mu = jnp.mean(out, axis=-1, keepdims=True)
    var = jnp.mean((out - mu) ** 2, axis=-1, keepdims=True)
    out_ref[...] = (out - mu) * jax.lax.rsqrt(var + 1e-5)


def _gat_layer(x, edge_index, W, al, ar, Wr, b, agg_mode, use_elu):
    n = x.shape[0]
    feat = (x @ W).reshape(n, K, H)
    el = jnp.sum(feat * al[None, :, :], axis=-1)
    er = jnp.sum(feat * ar[None, :, :], axis=-1)
    src = edge_index[0]
    dst = edge_index[1]
    e = el[src] + er[dst]
    e = jax.nn.leaky_relu(e, negative_slope=0.2)
    emax = jax.ops.segment_max(e, dst, num_segments=n)
    emax = jnp.where(jnp.isfinite(emax), emax, 0.0)
    ex = jnp.exp(e - emax[dst])
    denom = jax.ops.segment_sum(ex, dst, num_segments=n)
    alpha = ex / denom[dst]
    msg = feat[src] * alpha[:, :, None]
    rst = jax.ops.segment_sum(msg, dst, num_segments=n)
    rst = rst + (x @ Wr).reshape(n, K, H)
    rst = rst + b.reshape(1, K, H)
    if agg_mode == 'flatten':
        out = rst.reshape(n, K * H)
    else:
        out = rst.mean(axis=1)
    if use_elu:
        out = jax.nn.elu(out)
    return out


def kernel(x, edge_index, graph_ids, W1, al1, ar1, Wr1, b1, W2, al2, ar2,
           Wr2, b2, atom_w, atom_b, Wl, bl):
    h = _gat_layer(x, edge_index, W1, al1, ar1, Wr1, b1, 'flatten', True)
    h = _gat_layer(h, edge_index, W2, al2, ar2, Wr2, b2, 'mean', False)
    w = jax.nn.sigmoid(h @ atom_w + atom_b)
    sum_g = jax.ops.segment_sum(h * w, graph_ids, num_segments=G)
    max_g = jax.ops.segment_max(h, graph_ids, num_segments=G)
    max_g = jnp.where(jnp.isfinite(max_g), max_g, 0.0)
    gfeat = jnp.concatenate([sum_g, max_g], axis=1)
    out = pl.pallas_call(
        _head_body,
        out_shape=jax.ShapeDtypeStruct((G, 128), jnp.float32),
    )(gfeat, Wl, bl.reshape(1, 128))
    return out


# speedup vs baseline: 15.0089x; 15.0089x over previous
import functools

import jax
import jax.numpy as jnp
from jax import lax
from jax.experimental import pallas as pl
from jax.experimental.pallas import tpu as pltpu
from jax.experimental.pallas import tpu_sc as plsc

N = 50000
E = 800000
G = 256
K = 4
H = 32

EROWS = 6400
EPAD = EROWS * 128 - E
NPAD = 50048
NTILE = NPAD // 16
E1_RPW = EROWS // 32
E2_RPT = EROWS // 16
E2_CH = 40
NR = 50176
RID = 1568
RCH = 392
GP = 264

@functools.cache
def _mesh():
    return plsc.VectorSubcoreMesh(core_axis_name="c", subcore_axis_name="s")


def _tca_body(x_ref, w_ref, aler_ref, wr_ref, b_ref,
              f0, f1, f2, f3, eler_ref, res_ref):
    xb = x_ref[...]
    feat = jnp.dot(xb, w_ref[...], preferred_element_type=jnp.float32)
    eler_ref[...] = jnp.dot(feat, aler_ref[...],
                            preferred_element_type=jnp.float32)
    res_ref[...] = jnp.dot(xb, wr_ref[...],
                           preferred_element_type=jnp.float32) + b_ref[...]
    f0[...] = feat[:, 0:32]
    f1[...] = feat[:, 32:64]
    f2[...] = feat[:, 64:96]
    f3[...] = feat[:, 96:128]


def _tcb_body(a0, a1, a2, a3, den_ref, res_ref, w_ref, aler_ref,
              wrm_ref, bm_ref,
              f0, f1, f2, f3, eler_ref, resm_ref):
    den = den_ref[...]
    dsum = den[0] + den[1]
    dsafe = jnp.where(dsum > 0, dsum, 1.0)
    zs = [a_ref[...] / dsafe[:, k:k + 1]
          for k, a_ref in enumerate((a0, a1, a2, a3))]
    h = jnp.concatenate(zs, axis=1) + res_ref[...]
    h = jnp.where(h > 0, h, jnp.exp(h) - 1.0)
    feat = jnp.dot(h, w_ref[...], preferred_element_type=jnp.float32)
    eler_ref[...] = jnp.dot(feat, aler_ref[...],
                            preferred_element_type=jnp.float32)
    resm_ref[...] = jnp.dot(h, wrm_ref[...],
                            preferred_element_type=jnp.float32) + bm_ref[...]
    f0[...] = feat[:, 0:32]
    f1[...] = feat[:, 32:64]
    f2[...] = feat[:, 64:96]
    f3[...] = feat[:, 96:128]


def _tcc_body(a0, a1, a2, a3, den_ref, resm_ref, aw_ref, ab_ref,
              h2_ref, hw_ref):
    den = den_ref[...]
    dsum = den[0] + den[1]
    dsafe = jnp.where(dsum > 0, dsum, 1.0)
    acc = (a0[...] / dsafe[:, 0:1] + a1[...] / dsafe[:, 1:2]
           + a2[...] / dsafe[:, 2:3] + a3[...] / dsafe[:, 3:4])
    h2 = acc * 0.25 + resm_ref[...]
    wv = jax.nn.sigmoid(jnp.sum(h2 * aw_ref[...], axis=1, keepdims=True)
                        + ab_ref[...])
    h2_ref[...] = h2
    hw_ref[...] = h2 * wv


def _tcd_body(sums_ref, maxs_ref, wl_ref, bl_ref, out_ref):
    s = jnp.sum(sums_ref[...][:, :G, :], axis=0)
    m = jnp.max(maxs_ref[...][:, :G, :], axis=0)
    m = jnp.where(jnp.isfinite(m), m, 0.0)
    gfeat = jnp.concatenate([s, m], axis=1)
    out = jnp.dot(gfeat, wl_ref[...],
                  preferred_element_type=jnp.float32) + bl_ref[...]
    mu = jnp.mean(out, axis=-1, keepdims=True)
    var = jnp.mean((out - mu) ** 2, axis=-1, keepdims=True)
    out_ref[...] = (out - mu) * lax.rsqrt(var + 1e-5)


_R = 2000
_GRID = N // _R


def _tc_a(x, W, ALER, Wr, b):
    return pl.pallas_call(
        _tca_body,
        grid=(_GRID,),
        in_specs=[
            pl.BlockSpec((_R, 27), lambda i: (i, 0)),
            pl.BlockSpec((27, 128), lambda i: (0, 0)),
            pl.BlockSpec((128, 8), lambda i: (0, 0)),
            pl.BlockSpec((27, 128), lambda i: (0, 0)),
            pl.BlockSpec((1, 128), lambda i: (0, 0)),
        ],
        out_specs=[pl.BlockSpec((_R, 32), lambda i: (i, 0))] * 4
        + [pl.BlockSpec((_R, 8), lambda i: (i, 0)),
           pl.BlockSpec((_R, 128), lambda i: (i, 0))],
        out_shape=[jax.ShapeDtypeStruct((N, 32), jnp.float32)] * 4
        + [jax.ShapeDtypeStruct((N, 8), jnp.float32),
           jax.ShapeDtypeStruct((N, 128), jnp.float32)],
    )(x, W, ALER, Wr, b)


def _tc_b(a0, a1, a2, a3, denp, res1, W2, ALER2, Wr2m, b2m):
    return pl.pallas_call(
        _tcb_body,
        grid=(_GRID,),
        in_specs=[pl.BlockSpec((_R, 32), lambda i: (i, 0))] * 4
        + [
            pl.BlockSpec((2, _R, 16), lambda i: (0, i, 0)),
            pl.BlockSpec((_R, 128), lambda i: (i, 0)),
            pl.BlockSpec((128, 128), lambda i: (0, 0)),
            pl.BlockSpec((128, 8), lambda i: (0, 0)),
            pl.BlockSpec((128, 32), lambda i: (0, 0)),
            pl.BlockSpec((1, 32), lambda i: (0, 0)),
        ],
        out_specs=[pl.BlockSpec((_R, 32), lambda i: (i, 0))] * 4
        + [pl.BlockSpec((_R, 8), lambda i: (i, 0)),
           pl.BlockSpec((_R, 32), lambda i: (i, 0))],
        out_shape=[jax.ShapeDtypeStruct((N, 32), jnp.float32)] * 4
        + [jax.ShapeDtypeStruct((N, 8), jnp.float32),
           jax.ShapeDtypeStruct((N, 32), jnp.float32)],
    )(a0, a1, a2, a3, denp, res1, W2, ALER2, Wr2m, b2m)


def _tc_c(b0, b1, b2, b3, denp, res2m, aw, ab):
    return pl.pallas_call(
        _tcc_body,
        grid=(_GRID,),
        in_specs=[pl.BlockSpec((_R, 32), lambda i: (i, 0))] * 4
        + [
            pl.BlockSpec((2, _R, 16), lambda i: (0, i, 0)),
            pl.BlockSpec((_R, 32), lambda i: (i, 0)),
            pl.BlockSpec((1, 32), lambda i: (0, 0)),
            pl.BlockSpec((1, 1), lambda i: (0, 0)),
        ],
        out_specs=[pl.BlockSpec((_R, 32), lambda i: (i, 0))] * 2,
        out_shape=[jax.ShapeDtypeStruct((NR, 32), jnp.float32)] * 2,
    )(b0, b1, b2, b3, denp, res2m, aw, ab)


def _tc_d(sums, maxs, Wl, bl):
    return pl.pallas_call(
        _tcd_body,
        out_shape=jax.ShapeDtypeStruct((G, 128), jnp.float32),
    )(sums, maxs, Wl, bl)


_IOTA16 = None


def _sc_e1_body(eler_h, srcp_h, dstp_h, z16_h, exq_h, denp_h,
                den_sp, srcb, dstb, Sb, Db, EXb, EXbD):
    c = lax.axis_index("c")
    s = lax.axis_index("s")
    wid = s * 2 + c
    pltpu.sync_copy(z16_h.at[pl.ds(s * NTILE, NTILE)],
                    den_sp.at[pl.ds(s * NTILE, NTILE)])
    base = wid * E1_RPW
    pltpu.sync_copy(srcp_h.at[pl.ds(base, E1_RPW)], srcb)
    pltpu.sync_copy(dstp_h.at[pl.ds(base, E1_RPW)], dstb)
    iota = lax.iota(jnp.int32, 16)
    zero16 = jnp.zeros((16,), jnp.float32)

    def zinit(i, carry):
        EXbD[i, 0:16] = zero16
        return carry

    lax.fori_loop(0, 128, zinit, 0)
    plsc.subcore_barrier()

    def row_body(j, carry):
        pltpu.sync_copy(eler_h.at[srcb.at[j]], Sb)
        pltpu.sync_copy(eler_h.at[dstb.at[j]], Db)

        def grp(g, carry2):
            rows = iota + g * 16
            for k in range(4):
                colk = jnp.full((16,), k, jnp.int32)
                el = plsc.load_gather(Sb, [rows, colk])
                er = plsc.load_gather(Db, [rows, colk + 4])
                e = el + er
                e = jnp.where(e >= 0, e, e * 0.2)
                ex = jnp.exp(e)
                plsc.store_scatter(EXb, [rows, colk], ex)
                plsc.store_scatter(EXbD, [rows, colk], ex)
            return carry2

        lax.fori_loop(0, 8, grp, 0)
        pltpu.sync_copy(EXb, exq_h.at[base + j])
        pltpu.sync_copy(EXbD, den_sp.at[dstb.at[j]], add=True)
        return carry

    lax.fori_loop(0, E1_RPW, row_body, 0)
    plsc.subcore_barrier()
    pltpu.sync_copy(den_sp.at[pl.ds(s * NTILE, NTILE)],
                    denp_h.at[c, pl.ds(s * NTILE, NTILE)])


def _sc_e1(elerp, srcp, dstp, zeros16):
    f = functools.partial(
        pl.kernel,
        out_type=[jax.ShapeDtypeStruct((EROWS, 128, 4), jnp.float32),
                  jax.ShapeDtypeStruct((2, NPAD, 16), jnp.float32)],
        mesh=_mesh(),
        compiler_params=pltpu.CompilerParams(needs_layout_passes=False, use_tc_tiling_on_sc=False),
        scratch_types=[
            pltpu.VMEM_SHARED((NPAD, 16), jnp.float32),
            pltpu.VMEM((E1_RPW, 128), jnp.int32),
            pltpu.VMEM((E1_RPW, 128), jnp.int32),
            pltpu.VMEM((128, 8), jnp.float32),
            pltpu.VMEM((128, 8), jnp.float32),
            pltpu.VMEM((128, 4), jnp.float32),
            pltpu.VMEM((128, 16), jnp.float32),
        ],
    )(_sc_e1_body)
    return f(elerp, srcp, dstp, zeros16)


def _sc_e2_body(f0, f1, f2, f3, srcp_h, dstp_h, exq_h, z32_h,
                g0, g1, g2, g3,
                acc_sp, srcb, dstb, Fb, EXb, Mb):
    c = lax.axis_index("c")
    s = lax.axis_index("s")
    tbase = s * E2_RPT
    iota = lax.iota(jnp.int32, 16)

    ftabs = (f0, f1, f2, f3)
    gtabs = (g0, g1, g2, g3)

    for p in range(2):
        pltpu.sync_copy(z32_h.at[pl.ds(s * NTILE, NTILE)],
                        acc_sp.at[pl.ds(s * NTILE, NTILE)])
        plsc.subcore_barrier()
        for ci in range(2):
            khead = 2 * p + ci

            @pl.when(c == ci)
            def _(khead=khead):
                ftab = ftabs[khead]
                kcol = jnp.full((16,), khead, jnp.int32)

                def chunk_body(cc, carry0):
                    pltpu.sync_copy(
                        srcp_h.at[pl.ds(tbase + cc * E2_CH, E2_CH)], srcb)
                    pltpu.sync_copy(
                        dstp_h.at[pl.ds(tbase + cc * E2_CH, E2_CH)], dstb)

                    def row_body(jj, carry):
                        pltpu.sync_copy(ftab.at[srcb.at[jj]], Fb)
                        pltpu.sync_copy(exq_h.at[tbase + cc * E2_CH + jj],
                                        EXb)

                        def grp(g, carry2):
                            rows = iota + g * 16
                            ex16 = plsc.load_gather(EXb, [rows, kcol])
                            for fcol in range(32):
                                cols = jnp.full((16,), fcol, jnp.int32)
                                v = plsc.load_gather(Fb, [rows, cols])
                                plsc.store_scatter(Mb, [rows, cols],
                                                   v * ex16)
                            return carry2

                        lax.fori_loop(0, 8, grp, 0)
                        pltpu.sync_copy(Mb, acc_sp.at[dstb.at[jj]], add=True)
                        return carry

                    lax.fori_loop(0, E2_CH, row_body, 0)
                    return carry0

                lax.fori_loop(0, E2_RPT // E2_CH, chunk_body, 0)

        plsc.subcore_barrier()
        for ci in range(2):
            khead = 2 * p + ci

            @pl.when(c == ci)
            def _(khead=khead):
                pltpu.sync_copy(acc_sp.at[pl.ds(s * NTILE, NTILE)],
                                gtabs[khead].at[pl.ds(s * NTILE, NTILE)])

        plsc.subcore_barrier()


def _sc_e2(f0, f1, f2, f3, srcp, dstp, exq, zeros32):
    f = functools.partial(
        pl.kernel,
        out_type=[jax.ShapeDtypeStruct((NPAD, 32), jnp.float32)] * 4,
        mesh=_mesh(),
        compiler_params=pltpu.CompilerParams(needs_layout_passes=False, use_tc_tiling_on_sc=False),
        scratch_types=[
            pltpu.VMEM_SHARED((NPAD, 32), jnp.float32),
            pltpu.VMEM((E2_CH, 128), jnp.int32),
            pltpu.VMEM((E2_CH, 128), jnp.int32),
            pltpu.VMEM((128, 32), jnp.float32),
            pltpu.VMEM((128, 4), jnp.float32),
            pltpu.VMEM((128, 32), jnp.float32),
        ],
    )(_sc_e2_body)
    return f(f0, f1, f2, f3, srcp, dstp, exq, zeros32)


def _sc_r_body(h2_h, hw_h, gid_h, sums_h, maxs_h,
               gidb, hb, hwb, sacc, macc):
    c = lax.axis_index("c")
    s = lax.axis_index("s")
    w = s * 2 + c
    pltpu.sync_copy(gid_h.at[w], gidb.at[:, 0:RID])
    zero16 = jnp.zeros((16,), jnp.float32)
    ninf16 = jnp.full((16,), -jnp.inf, jnp.float32)

    def zinit(i, carry):
        sacc[i, 0:16] = zero16
        sacc[i, 16:32] = zero16
        macc[i, 0:16] = ninf16
        macc[i, 16:32] = ninf16
        return carry

    lax.fori_loop(0, GP, zinit, 0)
    base = w * RID
    for ch in range(RID // RCH):
        pltpu.sync_copy(h2_h.at[pl.ds(base + ch * RCH, RCH)], hb)
        pltpu.sync_copy(hw_h.at[pl.ds(base + ch * RCH, RCH)], hwb)

        def nb(i, carry, ch=ch):
            gv = gidb[0, pl.ds(ch * RCH + i, 16)]
            g = gv[0]
            h0 = hb[i, 0:16]
            h1 = hb[i, 16:32]
            macc[g, 0:16] = jnp.maximum(macc[g, 0:16], h0)
            macc[g, 16:32] = jnp.maximum(macc[g, 16:32], h1)
            sacc[g, 0:16] = sacc[g, 0:16] + hwb[i, 0:16]
            sacc[g, 16:32] = sacc[g, 16:32] + hwb[i, 16:32]
            return carry

        lax.fori_loop(0, RCH, nb, 0)
    pltpu.sync_copy(sacc, sums_h.at[w])
    pltpu.sync_copy(macc, maxs_h.at[w])


def _sc_r(h2t, hwt, gidp):
    f = functools.partial(
        pl.kernel,
        out_type=[jax.ShapeDtypeStruct((32, GP, 32), jnp.float32)] * 2,
        mesh=_mesh(),
        compiler_params=pltpu.CompilerParams(needs_layout_passes=False, use_tc_tiling_on_sc=False),
        scratch_types=[
            pltpu.VMEM((1, RID + 16), jnp.int32),
            pltpu.VMEM((RCH, 32), jnp.float32),
            pltpu.VMEM((RCH, 32), jnp.float32),
            pltpu.VMEM((GP, 32), jnp.float32),
            pltpu.VMEM((GP, 32), jnp.float32),
        ],
    )(_sc_r_body)
    return f(h2t, hwt, gidp)


def _make_aler(al, ar):
    blockid = jnp.repeat(jnp.arange(K), H)
    onehot = (blockid[:, None] == jnp.arange(K)[None, :]).astype(jnp.float32)
    return jnp.concatenate([al.reshape(K * H)[:, None] * onehot,
                            ar.reshape(K * H)[:, None] * onehot], axis=1)


def kernel(x, edge_index, graph_ids, W1, al1, ar1, Wr1, b1, W2, al2, ar2,
           Wr2, b2, atom_w, atom_b, Wl, bl):
    src = edge_index[0]
    dst = edge_index[1]
    srcp = jnp.concatenate(
        [src, jnp.zeros((EPAD,), jnp.int32)]).reshape(EROWS, 128)
    dstp = jnp.concatenate(
        [dst, jnp.full((EPAD,), N, jnp.int32)]).reshape(EROWS, 128)
    zeros16 = jnp.zeros((NPAD, 16), jnp.float32)
    zeros32 = jnp.zeros((NPAD, 32), jnp.float32)
    ALER1 = _make_aler(al1, ar1)
    ALER2 = _make_aler(al2, ar2)
    Wr2m = Wr2.reshape(K * H, K, H).mean(axis=1)
    b2m = b2.reshape(K, H).mean(axis=0).reshape(1, H)

    f0, f1, f2, f3, eler1, res1 = _tc_a(x, W1, ALER1, Wr1, b1.reshape(1, 128))
    eler1p = jnp.pad(eler1, ((0, NPAD - N), (0, 0)))
    exq1, denp1 = _sc_e1(eler1p, srcp, dstp, zeros16)
    a0, a1, a2, a3 = _sc_e2(f0, f1, f2, f3, srcp, dstp, exq1, zeros32)

    f20, f21, f22, f23, eler2, res2m = _tc_b(
        a0, a1, a2, a3, denp1, res1, W2, ALER2, Wr2m, b2m)
    eler2p = jnp.pad(eler2, ((0, NPAD - N), (0, 0)))
    exq2, denp2 = _sc_e1(eler2p, srcp, dstp, zeros16)
    c0, c1, c2, c3 = _sc_e2(f20, f21, f22, f23, srcp, dstp, exq2, zeros32)

    h2t, hwt = _tc_c(c0, c1, c2, c3, denp2, res2m,
                     atom_w.reshape(1, H), atom_b.reshape(1, 1))

    gidp = jnp.concatenate(
        [graph_ids, jnp.full((NR - N,), G, jnp.int32)]).reshape(32, 1, RID)
    sums, maxs = _sc_r(h2t, hwt, gidp)
    return _tc_d(sums, maxs, Wl, bl.reshape(1, 128))


# speedup vs baseline: 19.9198x; 1.3272x over previous
import functools

import jax
import jax.numpy as jnp
from jax import lax
from jax.experimental import pallas as pl
from jax.experimental.pallas import tpu as pltpu
from jax.experimental.pallas import tpu_sc as plsc

N = 50000
E = 800000
G = 256
K = 4
H = 32

EROWS = 6400
EPAD = EROWS * 128 - E
NPAD = 50048
NTILE = NPAD // 16
E1_RPW = EROWS // 32
E2_RPT = EROWS // 16
E2_CH = 40
NR = 50176
RID = 1568
RCH = 392
GP = 264

@functools.cache
def _mesh():
    return plsc.VectorSubcoreMesh(core_axis_name="c", subcore_axis_name="s")


def _tca_body(x_ref, w_ref, aler_ref, wr_ref, b_ref,
              f0, f1, f2, f3, eler_ref, res_ref):
    xb = x_ref[...]
    feat = jnp.dot(xb, w_ref[...], preferred_element_type=jnp.float32)
    eler_ref[...] = jnp.dot(feat, aler_ref[...],
                            preferred_element_type=jnp.float32)
    res_ref[...] = jnp.dot(xb, wr_ref[...],
                           preferred_element_type=jnp.float32) + b_ref[...]
    f0[...] = feat[:, 0:32]
    f1[...] = feat[:, 32:64]
    f2[...] = feat[:, 64:96]
    f3[...] = feat[:, 96:128]


def _tcb_body(a0, a1, a2, a3, den_ref, res_ref, w_ref, aler_ref,
              wrm_ref, bm_ref,
              f0, f1, f2, f3, eler_ref, resm_ref):
    den = den_ref[...]
    dsum = den[0] + den[1]
    dsafe = jnp.where(dsum > 0, dsum, 1.0)
    zs = [a_ref[...] / dsafe[:, k:k + 1]
          for k, a_ref in enumerate((a0, a1, a2, a3))]
    h = jnp.concatenate(zs, axis=1) + res_ref[...]
    h = jnp.where(h > 0, h, jnp.exp(h) - 1.0)
    feat = jnp.dot(h, w_ref[...], preferred_element_type=jnp.float32)
    eler_ref[...] = jnp.dot(feat, aler_ref[...],
                            preferred_element_type=jnp.float32)
    resm_ref[...] = jnp.dot(h, wrm_ref[...],
                            preferred_element_type=jnp.float32) + bm_ref[...]
    f0[...] = feat[:, 0:32]
    f1[...] = feat[:, 32:64]
    f2[...] = feat[:, 64:96]
    f3[...] = feat[:, 96:128]


def _tcc_body(a0, a1, a2, a3, den_ref, resm_ref, aw_ref, ab_ref,
              h2_ref, hw_ref):
    den = den_ref[...]
    dsum = den[0] + den[1]
    dsafe = jnp.where(dsum > 0, dsum, 1.0)
    acc = (a0[...] / dsafe[:, 0:1] + a1[...] / dsafe[:, 1:2]
           + a2[...] / dsafe[:, 2:3] + a3[...] / dsafe[:, 3:4])
    h2 = acc * 0.25 + resm_ref[...]
    wv = jax.nn.sigmoid(jnp.sum(h2 * aw_ref[...], axis=1, keepdims=True)
                        + ab_ref[...])
    h2_ref[...] = h2
    hw_ref[...] = h2 * wv


def _tcd_body(sums_ref, maxs_ref, wl_ref, bl_ref, out_ref):
    s = jnp.sum(sums_ref[...][:, :G, :], axis=0)
    m = jnp.max(maxs_ref[...][:, :G, :], axis=0)
    m = jnp.where(jnp.isfinite(m), m, 0.0)
    gfeat = jnp.concatenate([s, m], axis=1)
    out = jnp.dot(gfeat, wl_ref[...],
                  preferred_element_type=jnp.float32) + bl_ref[...]
    mu = jnp.mean(out, axis=-1, keepdims=True)
    var = jnp.mean((out - mu) ** 2, axis=-1, keepdims=True)
    out_ref[...] = (out - mu) * lax.rsqrt(var + 1e-5)


_R = 2000
_GRID = N // _R


def _tc_a(x, W, ALER, Wr, b):
    return pl.pallas_call(
        _tca_body,
        grid=(_GRID,),
        in_specs=[
            pl.BlockSpec((_R, 27), lambda i: (i, 0)),
            pl.BlockSpec((27, 128), lambda i: (0, 0)),
            pl.BlockSpec((128, 8), lambda i: (0, 0)),
            pl.BlockSpec((27, 128), lambda i: (0, 0)),
            pl.BlockSpec((1, 128), lambda i: (0, 0)),
        ],
        out_specs=[pl.BlockSpec((_R, 32), lambda i: (i, 0))] * 4
        + [pl.BlockSpec((_R, 8), lambda i: (i, 0)),
           pl.BlockSpec((_R, 128), lambda i: (i, 0))],
        out_shape=[jax.ShapeDtypeStruct((N, 32), jnp.float32)] * 4
        + [jax.ShapeDtypeStruct((N, 8), jnp.float32),
           jax.ShapeDtypeStruct((N, 128), jnp.float32)],
    )(x, W, ALER, Wr, b)


def _tc_b(a0, a1, a2, a3, denp, res1, W2, ALER2, Wr2m, b2m):
    return pl.pallas_call(
        _tcb_body,
        grid=(_GRID,),
        in_specs=[pl.BlockSpec((_R, 32), lambda i: (i, 0))] * 4
        + [
            pl.BlockSpec((2, _R, 16), lambda i: (0, i, 0)),
            pl.BlockSpec((_R, 128), lambda i: (i, 0)),
            pl.BlockSpec((128, 128), lambda i: (0, 0)),
            pl.BlockSpec((128, 8), lambda i: (0, 0)),
            pl.BlockSpec((128, 32), lambda i: (0, 0)),
            pl.BlockSpec((1, 32), lambda i: (0, 0)),
        ],
        out_specs=[pl.BlockSpec((_R, 32), lambda i: (i, 0))] * 4
        + [pl.BlockSpec((_R, 8), lambda i: (i, 0)),
           pl.BlockSpec((_R, 32), lambda i: (i, 0))],
        out_shape=[jax.ShapeDtypeStruct((N, 32), jnp.float32)] * 4
        + [jax.ShapeDtypeStruct((N, 8), jnp.float32),
           jax.ShapeDtypeStruct((N, 32), jnp.float32)],
    )(a0, a1, a2, a3, denp, res1, W2, ALER2, Wr2m, b2m)


def _tc_c(b0, b1, b2, b3, denp, res2m, aw, ab):
    return pl.pallas_call(
        _tcc_body,
        grid=(_GRID,),
        in_specs=[pl.BlockSpec((_R, 32), lambda i: (i, 0))] * 4
        + [
            pl.BlockSpec((2, _R, 16), lambda i: (0, i, 0)),
            pl.BlockSpec((_R, 32), lambda i: (i, 0)),
            pl.BlockSpec((1, 32), lambda i: (0, 0)),
            pl.BlockSpec((1, 1), lambda i: (0, 0)),
        ],
        out_specs=[pl.BlockSpec((_R, 32), lambda i: (i, 0))] * 2,
        out_shape=[jax.ShapeDtypeStruct((NR, 32), jnp.float32)] * 2,
    )(b0, b1, b2, b3, denp, res2m, aw, ab)


def _tc_d(sums, maxs, Wl, bl):
    return pl.pallas_call(
        _tcd_body,
        out_shape=jax.ShapeDtypeStruct((G, 128), jnp.float32),
    )(sums, maxs, Wl, bl)


_IOTA16 = None


def _sc_e1_body(eler_h, srcp_h, dstp_h, z16_h, exq_h, denp_h,
                den_sp, srcb, dstb, Sb, Db, EXb, EXbD):
    c = lax.axis_index("c")
    s = lax.axis_index("s")
    wid = s * 2 + c
    pltpu.sync_copy(z16_h.at[pl.ds(s * NTILE, NTILE)],
                    den_sp.at[pl.ds(s * NTILE, NTILE)])
    base = wid * E1_RPW
    pltpu.sync_copy(srcp_h.at[pl.ds(base, E1_RPW)], srcb)
    pltpu.sync_copy(dstp_h.at[pl.ds(base, E1_RPW)], dstb)
    iota = lax.iota(jnp.int32, 16)
    zero16 = jnp.zeros((16,), jnp.float32)

    def zinit(i, carry):
        EXbD[i, 0:16] = zero16
        return carry

    lax.fori_loop(0, 128, zinit, 0)
    plsc.subcore_barrier()

    def row_body(j, carry):
        pltpu.sync_copy(eler_h.at[srcb.at[j]], Sb)
        pltpu.sync_copy(eler_h.at[dstb.at[j]], Db)

        def grp(g, carry2):
            rows = iota + g * 16
            for k in range(4):
                colk = jnp.full((16,), k, jnp.int32)
                el = plsc.load_gather(Sb, [rows, colk])
                er = plsc.load_gather(Db, [rows, colk + 4])
                e = el + er
                e = jnp.where(e >= 0, e, e * 0.2)
                ex = jnp.exp(e)
                plsc.store_scatter(EXb, [rows, colk], ex)
                plsc.store_scatter(EXbD, [rows, colk], ex)
            return carry2

        lax.fori_loop(0, 8, grp, 0)
        pltpu.sync_copy(EXb, exq_h.at[base + j])
        pltpu.sync_copy(EXbD, den_sp.at[dstb.at[j]], add=True)
        return carry

    lax.fori_loop(0, E1_RPW, row_body, 0)
    plsc.subcore_barrier()
    pltpu.sync_copy(den_sp.at[pl.ds(s * NTILE, NTILE)],
                    denp_h.at[c, pl.ds(s * NTILE, NTILE)])


def _sc_e1(elerp, srcp, dstp, zeros16):
    f = functools.partial(
        pl.kernel,
        out_type=[jax.ShapeDtypeStruct((EROWS, 128, 4), jnp.float32),
                  jax.ShapeDtypeStruct((2, NPAD, 16), jnp.float32)],
        mesh=_mesh(),
        compiler_params=pltpu.CompilerParams(needs_layout_passes=False, use_tc_tiling_on_sc=False),
        scratch_types=[
            pltpu.VMEM_SHARED((NPAD, 16), jnp.float32),
            pltpu.VMEM((E1_RPW, 128), jnp.int32),
            pltpu.VMEM((E1_RPW, 128), jnp.int32),
            pltpu.VMEM((128, 8), jnp.float32),
            pltpu.VMEM((128, 8), jnp.float32),
            pltpu.VMEM((128, 4), jnp.float32),
            pltpu.VMEM((128, 16), jnp.float32),
        ],
    )(_sc_e1_body)
    return f(elerp, srcp, dstp, zeros16)


def _sc_e2_body(f0, f1, f2, f3, srcp_h, dstp_h, exq_h, z32_h,
                g0, g1, g2, g3,
                acc_sp, srcb, dstb, Fb0, Fb1, EXb0, EXb1, Mb0, Mb1,
                gsem0, gsem1, xsem0, xsem1, ssem0, ssem1):
    c = lax.axis_index("c")
    s = lax.axis_index("s")
    tbase = s * E2_RPT
    iota = lax.iota(jnp.int32, 16)

    ftabs = (f0, f1, f2, f3)
    gtabs = (g0, g1, g2, g3)

    for p in range(2):
        pltpu.sync_copy(z32_h.at[pl.ds(s * NTILE, NTILE)],
                        acc_sp.at[pl.ds(s * NTILE, NTILE)])
        plsc.subcore_barrier()
        for ci in range(2):
            khead = 2 * p + ci

            @pl.when(c == ci)
            def _(khead=khead):
                ftab = ftabs[khead]
                kcol = jnp.full((16,), khead, jnp.int32)

                def compute(Fb, EXb, Mb):
                    def grp(g, carry2):
                        rows = iota + g * 16
                        ex16 = plsc.load_gather(EXb, [rows, kcol])
                        for fcol in range(32):
                            cols = jnp.full((16,), fcol, jnp.int32)
                            v = plsc.load_gather(Fb, [rows, cols])
                            plsc.store_scatter(Mb, [rows, cols], v * ex16)
                        return carry2

                    lax.fori_loop(0, 8, grp, 0)

                def chunk_body(cc, carry0):
                    rbase = tbase + cc * E2_CH
                    pltpu.sync_copy(srcp_h.at[pl.ds(rbase, E2_CH)], srcb)
                    pltpu.sync_copy(dstp_h.at[pl.ds(rbase, E2_CH)], dstb)
                    pltpu.async_copy(ftab.at[srcb.at[0]], Fb0, gsem0)
                    pltpu.async_copy(exq_h.at[rbase], EXb0, xsem0)

                    def pair(jp, carry):
                        r0 = jp * 2
                        pltpu.async_copy(ftab.at[srcb.at[r0 + 1]], Fb1,
                                         gsem1)
                        pltpu.async_copy(exq_h.at[rbase + r0 + 1], EXb1,
                                         xsem1)
                        pltpu.make_async_copy(ftab.at[srcb.at[r0]], Fb0,
                                              gsem0).wait()
                        pltpu.make_async_copy(exq_h.at[rbase + r0], EXb0,
                                              xsem0).wait()

                        @pl.when(jp > 0)
                        def _():
                            pltpu.make_async_copy(
                                Mb0, acc_sp.at[dstb.at[r0 - 2]],
                                ssem0).wait()

                        compute(Fb0, EXb0, Mb0)
                        pltpu.async_copy(Mb0, acc_sp.at[dstb.at[r0]],
                                         ssem0, add=True)

                        @pl.when(jp < E2_CH // 2 - 1)
                        def _():
                            pltpu.async_copy(ftab.at[srcb.at[r0 + 2]], Fb0,
                                             gsem0)
                            pltpu.async_copy(exq_h.at[rbase + r0 + 2], EXb0,
                                             xsem0)

                        pltpu.make_async_copy(ftab.at[srcb.at[r0 + 1]], Fb1,
                                              gsem1).wait()
                        pltpu.make_async_copy(exq_h.at[rbase + r0 + 1], EXb1,
                                              xsem1).wait()

                        @pl.when(jp > 0)
                        def _():
                            pltpu.make_async_copy(
                                Mb1, acc_sp.at[dstb.at[r0 - 1]],
                                ssem1).wait()

                        compute(Fb1, EXb1, Mb1)
                        pltpu.async_copy(Mb1, acc_sp.at[dstb.at[r0 + 1]],
                                         ssem1, add=True)
                        return carry

                    lax.fori_loop(0, E2_CH // 2, pair, 0)
                    pltpu.make_async_copy(Mb0, acc_sp.at[dstb.at[E2_CH - 2]],
                                          ssem0).wait()
                    pltpu.make_async_copy(Mb1, acc_sp.at[dstb.at[E2_CH - 1]],
                                          ssem1).wait()
                    return carry0

                lax.fori_loop(0, E2_RPT // E2_CH, chunk_body, 0)

        plsc.subcore_barrier()
        for ci in range(2):
            khead = 2 * p + ci

            @pl.when(c == ci)
            def _(khead=khead):
                pltpu.sync_copy(acc_sp.at[pl.ds(s * NTILE, NTILE)],
                                gtabs[khead].at[pl.ds(s * NTILE, NTILE)])

        plsc.subcore_barrier()


def _sc_e2(f0, f1, f2, f3, srcp, dstp, exq, zeros32):
    f = functools.partial(
        pl.kernel,
        out_type=[jax.ShapeDtypeStruct((NPAD, 32), jnp.float32)] * 4,
        mesh=_mesh(),
        compiler_params=pltpu.CompilerParams(needs_layout_passes=False, use_tc_tiling_on_sc=False),
        scratch_types=[
            pltpu.VMEM_SHARED((NPAD, 32), jnp.float32),
            pltpu.VMEM((E2_CH, 128), jnp.int32),
            pltpu.VMEM((E2_CH, 128), jnp.int32),
            pltpu.VMEM((128, 32), jnp.float32),
            pltpu.VMEM((128, 32), jnp.float32),
            pltpu.VMEM((128, 4), jnp.float32),
            pltpu.VMEM((128, 4), jnp.float32),
            pltpu.VMEM((128, 32), jnp.float32),
            pltpu.VMEM((128, 32), jnp.float32),
            pltpu.SemaphoreType.DMA,
            pltpu.SemaphoreType.DMA,
            pltpu.SemaphoreType.DMA,
            pltpu.SemaphoreType.DMA,
            pltpu.SemaphoreType.DMA,
            pltpu.SemaphoreType.DMA,
        ],
    )(_sc_e2_body)
    return f(f0, f1, f2, f3, srcp, dstp, exq, zeros32)


def _sc_r_body(h2_h, hw_h, gid_h, sums_h, maxs_h,
               gidb, hb, hwb, sacc, macc):
    c = lax.axis_index("c")
    s = lax.axis_index("s")
    w = s * 2 + c
    pltpu.sync_copy(gid_h.at[w], gidb.at[:, 0:RID])
    zero16 = jnp.zeros((16,), jnp.float32)
    ninf16 = jnp.full((16,), -jnp.inf, jnp.float32)

    def zinit(i, carry):
        sacc[i, 0:16] = zero16
        sacc[i, 16:32] = zero16
        macc[i, 0:16] = ninf16
        macc[i, 16:32] = ninf16
        return carry

    lax.fori_loop(0, GP, zinit, 0)
    base = w * RID
    for ch in range(RID // RCH):
        pltpu.sync_copy(h2_h.at[pl.ds(base + ch * RCH, RCH)], hb)
        pltpu.sync_copy(hw_h.at[pl.ds(base + ch * RCH, RCH)], hwb)

        def nb(i, carry, ch=ch):
            gv = gidb[0, pl.ds(ch * RCH + i, 16)]
            g = gv[0]
            h0 = hb[i, 0:16]
            h1 = hb[i, 16:32]
            macc[g, 0:16] = jnp.maximum(macc[g, 0:16], h0)
            macc[g, 16:32] = jnp.maximum(macc[g, 16:32], h1)
            sacc[g, 0:16] = sacc[g, 0:16] + hwb[i, 0:16]
            sacc[g, 16:32] = sacc[g, 16:32] + hwb[i, 16:32]
            return carry

        lax.fori_loop(0, RCH, nb, 0)
    pltpu.sync_copy(sacc, sums_h.at[w])
    pltpu.sync_copy(macc, maxs_h.at[w])


def _sc_r(h2t, hwt, gidp):
    f = functools.partial(
        pl.kernel,
        out_type=[jax.ShapeDtypeStruct((32, GP, 32), jnp.float32)] * 2,
        mesh=_mesh(),
        compiler_params=pltpu.CompilerParams(needs_layout_passes=False, use_tc_tiling_on_sc=False),
        scratch_types=[
            pltpu.VMEM((1, RID + 16), jnp.int32),
            pltpu.VMEM((RCH, 32), jnp.float32),
            pltpu.VMEM((RCH, 32), jnp.float32),
            pltpu.VMEM((GP, 32), jnp.float32),
            pltpu.VMEM((GP, 32), jnp.float32),
        ],
    )(_sc_r_body)
    return f(h2t, hwt, gidp)


def _make_aler(al, ar):
    blockid = jnp.repeat(jnp.arange(K), H)
    onehot = (blockid[:, None] == jnp.arange(K)[None, :]).astype(jnp.float32)
    return jnp.concatenate([al.reshape(K * H)[:, None] * onehot,
                            ar.reshape(K * H)[:, None] * onehot], axis=1)


def kernel(x, edge_index, graph_ids, W1, al1, ar1, Wr1, b1, W2, al2, ar2,
           Wr2, b2, atom_w, atom_b, Wl, bl):
    src = edge_index[0]
    dst = edge_index[1]
    srcp = jnp.concatenate(
        [src, jnp.zeros((EPAD,), jnp.int32)]).reshape(EROWS, 128)
    dstp = jnp.concatenate(
        [dst, jnp.full((EPAD,), N, jnp.int32)]).reshape(EROWS, 128)
    zeros16 = jnp.zeros((NPAD, 16), jnp.float32)
    zeros32 = jnp.zeros((NPAD, 32), jnp.float32)
    ALER1 = _make_aler(al1, ar1)
    ALER2 = _make_aler(al2, ar2)
    Wr2m = Wr2.reshape(K * H, K, H).mean(axis=1)
    b2m = b2.reshape(K, H).mean(axis=0).reshape(1, H)

    f0, f1, f2, f3, eler1, res1 = _tc_a(x, W1, ALER1, Wr1, b1.reshape(1, 128))
    eler1p = jnp.pad(eler1, ((0, NPAD - N), (0, 0)))
    exq1, denp1 = _sc_e1(eler1p, srcp, dstp, zeros16)
    a0, a1, a2, a3 = _sc_e2(f0, f1, f2, f3, srcp, dstp, exq1, zeros32)

    f20, f21, f22, f23, eler2, res2m = _tc_b(
        a0, a1, a2, a3, denp1, res1, W2, ALER2, Wr2m, b2m)
    eler2p = jnp.pad(eler2, ((0, NPAD - N), (0, 0)))
    exq2, denp2 = _sc_e1(eler2p, srcp, dstp, zeros16)
    c0, c1, c2, c3 = _sc_e2(f20, f21, f22, f23, srcp, dstp, exq2, zeros32)

    h2t, hwt = _tc_c(c0, c1, c2, c3, denp2, res2m,
                     atom_w.reshape(1, H), atom_b.reshape(1, 1))

    gidp = jnp.concatenate(
        [graph_ids, jnp.full((NR - N,), G, jnp.int32)]).reshape(32, 1, RID)
    sums, maxs = _sc_r(h2t, hwt, gidp)
    return _tc_d(sums, maxs, Wl, bl.reshape(1, 128))
